# initial kernel scaffold (unmeasured)
import jax
import jax.numpy as jnp
from jax import lax
from jax.experimental import pallas as pl
from jax.experimental.pallas import tpu as pltpu


def kernel(
    x,
):
    def body(*refs):
        pass

    out_shape = jax.ShapeDtypeStruct(..., jnp.float32)
    return pl.pallas_call(body, out_shape=out_shape)(...)


# baseline (device time: 59447 ns/iter reference)
import jax
import jax.numpy as jnp
from jax import lax
from jax.experimental import pallas as pl
from jax.experimental.pallas import tpu as pltpu

N_DEV = 4


def kernel(x):
    m, n = x.shape
    n_per = n // N_DEV

    def body(x_ref, out_ref, stage_ref, send_sems, recv_sems):
        me = lax.axis_index("i")

        barrier_sem = pltpu.get_barrier_semaphore()
        for k in range(1, N_DEV):
            peer = (me + k) % N_DEV
            pl.semaphore_signal(
                barrier_sem, inc=1,
                device_id=(peer,), device_id_type=pl.DeviceIdType.MESH,
            )
        pl.semaphore_wait(barrier_sem, N_DEV - 1)

        out_ref[pl.ds(me * m, m), :] = x_ref[:, pl.ds(me * n_per, n_per)].astype(
            out_ref.dtype
        )

        sends = []
        for k in (1, 3, 2):
            dst = (me + k) % N_DEV
            slot = k - 1
            stage_ref[slot, :, :] = x_ref[:, pl.ds(dst * n_per, n_per)].astype(
                stage_ref.dtype
            )
            rdma = pltpu.make_async_remote_copy(
                src_ref=stage_ref.at[slot],
                dst_ref=out_ref.at[pl.ds(me * m, m)],
                send_sem=send_sems.at[slot],
                recv_sem=recv_sems.at[slot],
                device_id=(dst,),
                device_id_type=pl.DeviceIdType.MESH,
            )
            rdma.start()
            sends.append(rdma)

        for k in (1, 3, 2):
            origin = (me - k) % N_DEV
            recv = pltpu.make_async_remote_copy(
                src_ref=stage_ref.at[k - 1],
                dst_ref=out_ref.at[pl.ds(origin * m, m)],
                send_sem=send_sems.at[k - 1],
                recv_sem=recv_sems.at[k - 1],
                device_id=(me,),
                device_id_type=pl.DeviceIdType.MESH,
            )
            recv.wait_recv()

        for rdma in sends:
            rdma.wait_send()

    out_shape = jax.ShapeDtypeStruct((N_DEV * m, n_per), jnp.bfloat16)
    return pl.pallas_call(
        body,
        out_shape=out_shape,
        in_specs=[pl.BlockSpec(memory_space=pltpu.VMEM)],
        out_specs=pl.BlockSpec(memory_space=pltpu.VMEM),
        scratch_shapes=[
            pltpu.VMEM((N_DEV - 1, m, n_per), jnp.bfloat16),
            pltpu.SemaphoreType.DMA((N_DEV - 1,)),
            pltpu.SemaphoreType.DMA((N_DEV - 1,)),
        ],
        compiler_params=pltpu.CompilerParams(collective_id=0),
    )(x)


# device time: 56530 ns/iter; 1.0516x vs baseline; 1.0516x over previous
import jax
import jax.numpy as jnp
from jax import lax
from jax.experimental import pallas as pl
from jax.experimental.pallas import tpu as pltpu

N_DEV = 4


def kernel(x):
    m, n = x.shape
    n_per = n // N_DEV

    def body(x_ref, out_ref, xv_ref, stage_ref, cp_sems, send_sems, recv_sems):
        me = lax.axis_index("i")

        def chunk_load(k_or_me, buf):
            dst = (me + k_or_me) % N_DEV
            cp = pltpu.make_async_copy(
                x_ref.at[:, pl.ds(dst * n_per, n_per)],
                xv_ref.at[buf],
                cp_sems.at[buf],
            )
            cp.start()
            return cp

        cp1 = chunk_load(1, 0)
        cp3 = chunk_load(3, 1)

        barrier_sem = pltpu.get_barrier_semaphore()
        for k in range(1, N_DEV):
            peer = (me + k) % N_DEV
            pl.semaphore_signal(
                barrier_sem, inc=1,
                device_id=(peer,), device_id_type=pl.DeviceIdType.MESH,
            )
        pl.semaphore_wait(barrier_sem, N_DEV - 1)

        def send(k):
            dst = (me + k) % N_DEV
            slot = k - 1
            rdma = pltpu.make_async_remote_copy(
                src_ref=stage_ref.at[slot],
                dst_ref=out_ref.at[pl.ds(me * m, m)],
                send_sem=send_sems.at[slot],
                recv_sem=recv_sems.at[slot],
                device_id=(dst,),
                device_id_type=pl.DeviceIdType.MESH,
            )
            rdma.start()
            return rdma

        cp1.wait()
        stage_ref[0, :, :] = xv_ref[0].astype(stage_ref.dtype)
        s1 = send(1)
        cp2 = chunk_load(2, 0)

        cp3.wait()
        stage_ref[2, :, :] = xv_ref[1].astype(stage_ref.dtype)
        s3 = send(3)
        cp0 = chunk_load(0, 1)

        cp2.wait()
        stage_ref[1, :, :] = xv_ref[0].astype(stage_ref.dtype)
        s2 = send(2)

        cp0.wait()
        out_ref[pl.ds(me * m, m), :] = xv_ref[1].astype(out_ref.dtype)

        for k in (1, 3, 2):
            origin = (me - k) % N_DEV
            recv = pltpu.make_async_remote_copy(
                src_ref=stage_ref.at[k - 1],
                dst_ref=out_ref.at[pl.ds(origin * m, m)],
                send_sem=send_sems.at[k - 1],
                recv_sem=recv_sems.at[k - 1],
                device_id=(me,),
                device_id_type=pl.DeviceIdType.MESH,
            )
            recv.wait_recv()

        for rdma in (s1, s3, s2):
            rdma.wait_send()

    out_shape = jax.ShapeDtypeStruct((N_DEV * m, n_per), jnp.bfloat16)
    return pl.pallas_call(
        body,
        out_shape=out_shape,
        in_specs=[pl.BlockSpec(memory_space=pl.ANY)],
        out_specs=pl.BlockSpec(memory_space=pltpu.VMEM),
        scratch_shapes=[
            pltpu.VMEM((2, m, n_per), jnp.float32),
            pltpu.VMEM((N_DEV - 1, m, n_per), jnp.bfloat16),
            pltpu.SemaphoreType.DMA((2,)),
            pltpu.SemaphoreType.DMA((N_DEV - 1,)),
            pltpu.SemaphoreType.DMA((N_DEV - 1,)),
        ],
        compiler_params=pltpu.CompilerParams(collective_id=0),
    )(x)


# device time: 56516 ns/iter; 1.0519x vs baseline; 1.0002x over previous
import jax
import jax.numpy as jnp
from jax import lax
from jax.experimental import pallas as pl
from jax.experimental.pallas import tpu as pltpu

N_DEV = 4


def kernel(x):
    m, n = x.shape
    n_per = n // N_DEV

    def body(x_ref, out_ref, xv_ref, stage_ref, cp_sems, send_sems, recv_sems):
        me = lax.axis_index("i")

        def chunk_load(k_or_me, buf):
            dst = (me + k_or_me) % N_DEV
            cp = pltpu.make_async_copy(
                x_ref.at[:, pl.ds(dst * n_per, n_per)],
                xv_ref.at[buf],
                cp_sems.at[buf],
            )
            cp.start()
            return cp

        cp1 = chunk_load(1, 0)
        cp3 = chunk_load(3, 1)

        barrier_sem = pltpu.get_barrier_semaphore()
        for k in range(1, N_DEV):
            peer = (me + k) % N_DEV
            pl.semaphore_signal(
                barrier_sem, inc=1,
                device_id=(peer,), device_id_type=pl.DeviceIdType.MESH,
            )
        pl.semaphore_wait(barrier_sem, N_DEV - 1)

        def send(k):
            dst = (me + k) % N_DEV
            slot = k - 1
            rdma = pltpu.make_async_remote_copy(
                src_ref=stage_ref.at[slot],
                dst_ref=out_ref.at[pl.ds(me * m, m)],
                send_sem=send_sems.at[slot],
                recv_sem=recv_sems.at[slot],
                device_id=(dst,),
                device_id_type=pl.DeviceIdType.MESH,
            )
            rdma.start()
            return rdma

        cp1.wait()
        stage_ref[0, :, :] = xv_ref[0].astype(stage_ref.dtype)
        s1 = send(1)
        cp2 = chunk_load(2, 0)

        cp3.wait()
        stage_ref[2, :, :] = xv_ref[1].astype(stage_ref.dtype)
        s3 = send(3)
        cp0 = chunk_load(0, 1)

        cp2.wait()
        stage_ref[1, :, :] = xv_ref[0].astype(stage_ref.dtype)
        s2 = send(2)

        cp0.wait()
        stage_ref[N_DEV - 1, :, :] = xv_ref[1].astype(stage_ref.dtype)
        cp_local = pltpu.make_async_copy(
            stage_ref.at[N_DEV - 1],
            out_ref.at[pl.ds(me * m, m)],
            cp_sems.at[1],
        )
        cp_local.start()

        for k in (1, 3, 2):
            origin = (me - k) % N_DEV
            recv = pltpu.make_async_remote_copy(
                src_ref=stage_ref.at[k - 1],
                dst_ref=out_ref.at[pl.ds(origin * m, m)],
                send_sem=send_sems.at[k - 1],
                recv_sem=recv_sems.at[k - 1],
                device_id=(me,),
                device_id_type=pl.DeviceIdType.MESH,
            )
            recv.wait_recv()

        for rdma in (s1, s3, s2):
            rdma.wait_send()
        cp_local.wait()

    out_shape = jax.ShapeDtypeStruct((N_DEV * m, n_per), jnp.bfloat16)
    return pl.pallas_call(
        body,
        out_shape=out_shape,
        in_specs=[pl.BlockSpec(memory_space=pl.ANY)],
        out_specs=pl.BlockSpec(memory_space=pl.ANY),
        scratch_shapes=[
            pltpu.VMEM((2, m, n_per), jnp.float32),
            pltpu.VMEM((N_DEV, m, n_per), jnp.bfloat16),
            pltpu.SemaphoreType.DMA((2,)),
            pltpu.SemaphoreType.DMA((N_DEV - 1,)),
            pltpu.SemaphoreType.DMA((N_DEV - 1,)),
        ],
        compiler_params=pltpu.CompilerParams(collective_id=0),
    )(x)
